# baseline (device time: 342815 ns/iter reference)
import jax
import jax.numpy as jnp
from jax import lax
from jax.experimental import pallas as pl
from jax.experimental.pallas import tpu as pltpu

N_DEV = 8
N_SLOTS = 4


def kernel(x, w_mat, scale_x, scale_w):
    m, k_per = x.shape
    _, n = w_mat.shape
    m_per = m // N_DEV

    def body(x_ref, w_ref, sx_ref, sw_ref, out_ref,
             comm_ref, send_sems, recv_sems, credit_sem):
        my = lax.axis_index("i")
        left = lax.rem(my + N_DEV - 1, N_DEV)
        right = lax.rem(my + 1, N_DEV)

        barrier_sem = pltpu.get_barrier_semaphore()
        pl.semaphore_signal(barrier_sem, inc=1, device_id=(left,),
                            device_id_type=pl.DeviceIdType.MESH)
        pl.semaphore_signal(barrier_sem, inc=1, device_id=(right,),
                            device_id_type=pl.DeviceIdType.MESH)
        pl.semaphore_wait(barrier_sem, 2)

        w = w_ref[...].astype(jnp.bfloat16)

        def partial(c):
            xs = x_ref[pl.ds(c * m_per, m_per), :].astype(jnp.bfloat16)
            return lax.dot_general(xs, w, (((1,), (0,)), ((), ())),
                                   preferred_element_type=jnp.float32)

        comm_ref[0] = partial(lax.rem(my + N_DEV - 1, N_DEV))

        for s in range(N_DEV - 1):
            send_slot = s % N_SLOTS
            recv_slot = (s + 1) % N_SLOTS
            if s >= N_SLOTS - 1:
                pl.semaphore_wait(credit_sem, 1)
            rdma = pltpu.make_async_remote_copy(
                src_ref=comm_ref.at[send_slot],
                dst_ref=comm_ref.at[recv_slot],
                send_sem=send_sems.at[send_slot],
                recv_sem=recv_sems.at[recv_slot],
                device_id=(right,),
                device_id_type=pl.DeviceIdType.MESH,
            )
            rdma.start()
            c = lax.rem(my + 2 * N_DEV - 2 - s, N_DEV)
            p = partial(c)
            rdma.wait()
            if s < N_DEV - 2:
                comm_ref[recv_slot] = comm_ref[recv_slot] + p
            else:
                scale = sx_ref[0] * sw_ref[0]
                out_ref[...] = (comm_ref[recv_slot] + p) * scale
            if s <= (N_DEV - 2) - (N_SLOTS - 1):
                pl.semaphore_signal(credit_sem, inc=1, device_id=(left,),
                                    device_id_type=pl.DeviceIdType.MESH)

    return pl.pallas_call(
        body,
        out_shape=jax.ShapeDtypeStruct((m_per, n), jnp.float32),
        in_specs=[
            pl.BlockSpec(memory_space=pltpu.VMEM),
            pl.BlockSpec(memory_space=pltpu.VMEM),
            pl.BlockSpec(memory_space=pltpu.SMEM),
            pl.BlockSpec(memory_space=pltpu.SMEM),
        ],
        out_specs=pl.BlockSpec(memory_space=pltpu.VMEM),
        scratch_shapes=[
            pltpu.VMEM((N_SLOTS, m_per, n), jnp.float32),
            pltpu.SemaphoreType.DMA((N_SLOTS,)),
            pltpu.SemaphoreType.DMA((N_SLOTS,)),
            pltpu.SemaphoreType.REGULAR,
        ],
        compiler_params=pltpu.CompilerParams(collective_id=0),
    )(x, w_mat, scale_x, scale_w)


# device time: 185082 ns/iter; 1.8522x vs baseline; 1.8522x over previous
import jax
import jax.numpy as jnp
from jax import lax
from jax.experimental import pallas as pl
from jax.experimental.pallas import tpu as pltpu

N_DEV = 8
N_SLOTS = 4


def kernel(x, w_mat, scale_x, scale_w):
    m, k_per = x.shape
    _, n = w_mat.shape
    m_per = m // N_DEV

    def body(x_ref, w_ref, sx_ref, sw_ref, out_ref,
             comm_ref, send_sems, recv_sems, credit_sem):
        my = lax.axis_index("i")
        left = lax.rem(my + N_DEV - 1, N_DEV)
        right = lax.rem(my + 1, N_DEV)

        barrier_sem = pltpu.get_barrier_semaphore()
        pl.semaphore_signal(barrier_sem, inc=1, device_id=(left,),
                            device_id_type=pl.DeviceIdType.MESH)
        pl.semaphore_signal(barrier_sem, inc=1, device_id=(right,),
                            device_id_type=pl.DeviceIdType.MESH)
        pl.semaphore_wait(barrier_sem, 2)

        w = w_ref[...].astype(jnp.bfloat16)

        def partial(c):
            xs = x_ref[pl.ds(c * m_per, m_per), :].astype(jnp.bfloat16)
            return lax.dot_general(xs, w, (((1,), (0,)), ((), ())),
                                   preferred_element_type=jnp.float32)

        comm_ref[0] = partial(lax.rem(my + N_DEV - 1, N_DEV)).astype(jnp.bfloat16)

        for s in range(N_DEV - 1):
            send_slot = s % N_SLOTS
            recv_slot = (s + 1) % N_SLOTS
            if s >= N_SLOTS - 1:
                pl.semaphore_wait(credit_sem, 1)
            rdma = pltpu.make_async_remote_copy(
                src_ref=comm_ref.at[send_slot],
                dst_ref=comm_ref.at[recv_slot],
                send_sem=send_sems.at[send_slot],
                recv_sem=recv_sems.at[recv_slot],
                device_id=(right,),
                device_id_type=pl.DeviceIdType.MESH,
            )
            rdma.start()
            c = lax.rem(my + 2 * N_DEV - 2 - s, N_DEV)
            p = partial(c)
            rdma.wait()
            if s < N_DEV - 2:
                comm_ref[recv_slot] = (
                    comm_ref[recv_slot].astype(jnp.float32) + p
                ).astype(jnp.bfloat16)
            else:
                scale = sx_ref[0] * sw_ref[0]
                out_ref[...] = (
                    comm_ref[recv_slot].astype(jnp.float32) + p
                ) * scale
            if s <= (N_DEV - 2) - (N_SLOTS - 1):
                pl.semaphore_signal(credit_sem, inc=1, device_id=(left,),
                                    device_id_type=pl.DeviceIdType.MESH)

    return pl.pallas_call(
        body,
        out_shape=jax.ShapeDtypeStruct((m_per, n), jnp.float32),
        in_specs=[
            pl.BlockSpec(memory_space=pltpu.VMEM),
            pl.BlockSpec(memory_space=pltpu.VMEM),
            pl.BlockSpec(memory_space=pltpu.SMEM),
            pl.BlockSpec(memory_space=pltpu.SMEM),
        ],
        out_specs=pl.BlockSpec(memory_space=pltpu.VMEM),
        scratch_shapes=[
            pltpu.VMEM((N_SLOTS, m_per, n), jnp.bfloat16),
            pltpu.SemaphoreType.DMA((N_SLOTS,)),
            pltpu.SemaphoreType.DMA((N_SLOTS,)),
            pltpu.SemaphoreType.REGULAR,
        ],
        compiler_params=pltpu.CompilerParams(collective_id=0),
    )(x, w_mat, scale_x, scale_w)


# device time: 109407 ns/iter; 3.1334x vs baseline; 1.6917x over previous
import jax
import jax.numpy as jnp
from jax import lax
from jax.experimental import pallas as pl
from jax.experimental.pallas import tpu as pltpu

N_DEV = 8
N_SLOTS = 4


def kernel(x, w_mat, scale_x, scale_w):
    m, k_per = x.shape
    _, n = w_mat.shape
    m_per = m // N_DEV
    n_half = n // 2

    def body(x_ref, w_ref, sx_ref, sw_ref, out_ref,
             comm_r_ref, comm_l_ref, send_r_sems, recv_r_sems,
             send_l_sems, recv_l_sems, credit_r_sem, credit_l_sem):
        my = lax.axis_index("i")
        left = lax.rem(my + N_DEV - 1, N_DEV)
        right = lax.rem(my + 1, N_DEV)

        barrier_sem = pltpu.get_barrier_semaphore()
        pl.semaphore_signal(barrier_sem, inc=1, device_id=(left,),
                            device_id_type=pl.DeviceIdType.MESH)
        pl.semaphore_signal(barrier_sem, inc=1, device_id=(right,),
                            device_id_type=pl.DeviceIdType.MESH)
        pl.semaphore_wait(barrier_sem, 2)

        w = w_ref[...].astype(jnp.bfloat16)

        def partial(c, half):
            xs = x_ref[pl.ds(c * m_per, m_per), :].astype(jnp.bfloat16)
            wh = w[:, half * n_half:(half + 1) * n_half]
            return lax.dot_general(xs, wh, (((1,), (0,)), ((), ())),
                                   preferred_element_type=jnp.float32)

        comm_r_ref[0] = partial(lax.rem(my + N_DEV - 1, N_DEV), 0).astype(jnp.bfloat16)
        comm_l_ref[0] = partial(lax.rem(my + 1, N_DEV), 1).astype(jnp.bfloat16)

        for s in range(N_DEV - 1):
            send_slot = s % N_SLOTS
            recv_slot = (s + 1) % N_SLOTS
            if s >= N_SLOTS - 1:
                pl.semaphore_wait(credit_r_sem, 1)
                pl.semaphore_wait(credit_l_sem, 1)
            rdma_r = pltpu.make_async_remote_copy(
                src_ref=comm_r_ref.at[send_slot],
                dst_ref=comm_r_ref.at[recv_slot],
                send_sem=send_r_sems.at[send_slot],
                recv_sem=recv_r_sems.at[recv_slot],
                device_id=(right,),
                device_id_type=pl.DeviceIdType.MESH,
            )
            rdma_l = pltpu.make_async_remote_copy(
                src_ref=comm_l_ref.at[send_slot],
                dst_ref=comm_l_ref.at[recv_slot],
                send_sem=send_l_sems.at[send_slot],
                recv_sem=recv_l_sems.at[recv_slot],
                device_id=(left,),
                device_id_type=pl.DeviceIdType.MESH,
            )
            rdma_r.start()
            rdma_l.start()
            c_r = lax.rem(my + 2 * N_DEV - 2 - s, N_DEV)
            c_l = lax.rem(my + 2 + s, N_DEV)
            p_r = partial(c_r, 0)
            p_l = partial(c_l, 1)
            rdma_r.wait()
            rdma_l.wait()
            if s < N_DEV - 2:
                comm_r_ref[recv_slot] = (
                    comm_r_ref[recv_slot].astype(jnp.float32) + p_r
                ).astype(jnp.bfloat16)
                comm_l_ref[recv_slot] = (
                    comm_l_ref[recv_slot].astype(jnp.float32) + p_l
                ).astype(jnp.bfloat16)
            else:
                scale = sx_ref[0] * sw_ref[0]
                out_ref[:, :n_half] = (
                    comm_r_ref[recv_slot].astype(jnp.float32) + p_r
                ) * scale
                out_ref[:, n_half:] = (
                    comm_l_ref[recv_slot].astype(jnp.float32) + p_l
                ) * scale
            if s <= (N_DEV - 2) - (N_SLOTS - 1):
                pl.semaphore_signal(credit_r_sem, inc=1, device_id=(left,),
                                    device_id_type=pl.DeviceIdType.MESH)
                pl.semaphore_signal(credit_l_sem, inc=1, device_id=(right,),
                                    device_id_type=pl.DeviceIdType.MESH)

    return pl.pallas_call(
        body,
        out_shape=jax.ShapeDtypeStruct((m_per, n), jnp.float32),
        in_specs=[
            pl.BlockSpec(memory_space=pltpu.VMEM),
            pl.BlockSpec(memory_space=pltpu.VMEM),
            pl.BlockSpec(memory_space=pltpu.SMEM),
            pl.BlockSpec(memory_space=pltpu.SMEM),
        ],
        out_specs=pl.BlockSpec(memory_space=pltpu.VMEM),
        scratch_shapes=[
            pltpu.VMEM((N_SLOTS, m_per, n_half), jnp.bfloat16),
            pltpu.VMEM((N_SLOTS, m_per, n_half), jnp.bfloat16),
            pltpu.SemaphoreType.DMA((N_SLOTS,)),
            pltpu.SemaphoreType.DMA((N_SLOTS,)),
            pltpu.SemaphoreType.DMA((N_SLOTS,)),
            pltpu.SemaphoreType.DMA((N_SLOTS,)),
            pltpu.SemaphoreType.REGULAR,
            pltpu.SemaphoreType.REGULAR,
        ],
        compiler_params=pltpu.CompilerParams(collective_id=0),
    )(x, w_mat, scale_x, scale_w)


# device time: 81910 ns/iter; 4.1853x vs baseline; 1.3357x over previous
import jax
import jax.numpy as jnp
from jax import lax
from jax.experimental import pallas as pl
from jax.experimental.pallas import tpu as pltpu

N_DEV = 8
PERMS = ((1, 3, 4), (3, 4, 1), (4, 1, 3))
COLS = ((0, 768), (768, 1408), (1408, 2048))


def kernel(x, w_mat, scale_x, scale_w):
    m, k_per = x.shape
    _, n = w_mat.shape
    m_per = m // N_DEV

    def body(x_ref, w_ref, sx_ref, sw_ref, out_ref,
             acc0, acc1, acc2, r1_0, r1_1, r1_2, r2_0, r2_1, r2_2,
             r3_0, r3_1, r3_2, send_sems, recv_sems):
        accs = (acc0, acc1, acc2)
        recv1 = (r1_0, r1_1, r1_2)
        recv2 = (r2_0, r2_1, r2_2)
        recv3 = (r3_0, r3_1, r3_2)
        my = lax.axis_index("i")

        barrier_sem = pltpu.get_barrier_semaphore()
        for mask in (1, 3, 4):
            pl.semaphore_signal(barrier_sem, inc=1, device_id=(my ^ mask,),
                                device_id_type=pl.DeviceIdType.MESH)
        pl.semaphore_wait(barrier_sem, 3)

        w = w_ref[...].astype(jnp.bfloat16)

        def partial(c, g):
            xs = x_ref[pl.ds(c * m_per, m_per), :].astype(jnp.bfloat16)
            wg = w[:, COLS[g][0]:COLS[g][1]]
            return lax.dot_general(xs, wg, (((1,), (0,)), ((), ())),
                                   preferred_element_type=jnp.float32)

        def chunk_of(g, code):
            b1, b2, b3 = (code >> 2) & 1, (code >> 1) & 1, code & 1
            m1, m2, m3 = PERMS[g]
            return my ^ (b1 * m1 ^ b2 * m2 ^ b3 * m3)

        def exchange(g, step, src, dst):
            rdma = pltpu.make_async_remote_copy(
                src_ref=src, dst_ref=dst,
                send_sem=send_sems.at[3 * g + step],
                recv_sem=recv_sems.at[3 * g + step],
                device_id=(my ^ PERMS[g][step],),
                device_id_type=pl.DeviceIdType.MESH,
            )
            rdma.start()
            return rdma

        def add(a, b):
            return (a.astype(jnp.float32) + b.astype(jnp.float32)
                    ).astype(jnp.bfloat16)

        rd1 = []
        for g in range(3):
            for code in range(4, 8):
                accs[g][code] = partial(chunk_of(g, code), g).astype(jnp.bfloat16)
            rd1.append(exchange(g, 0, accs[g].at[4:8], recv1[g]))
        for g in range(3):
            for code in range(0, 4):
                accs[g][code] = partial(chunk_of(g, code), g).astype(jnp.bfloat16)

        rd2 = []
        for g in range(3):
            rd1[g].wait()
            accs[g][2:4] = add(accs[g][2:4], recv1[g][2:4])
            rd2.append(exchange(g, 1, accs[g].at[2:4], recv2[g]))
        for g in range(3):
            accs[g][0:2] = add(accs[g][0:2], recv1[g][0:2])

        rd3 = []
        for g in range(3):
            rd2[g].wait()
            accs[g][1] = add(accs[g][1], recv2[g][1])
            rd3.append(exchange(g, 2, accs[g].at[1], recv3[g]))
        for g in range(3):
            accs[g][0] = add(accs[g][0], recv2[g][0])

        scale = sx_ref[0] * sw_ref[0]
        for g in range(3):
            rd3[g].wait()
            out_ref[:, COLS[g][0]:COLS[g][1]] = (
                accs[g][0].astype(jnp.float32)
                + recv3[g][...].astype(jnp.float32)
            ) * scale

    widths = [hi - lo for lo, hi in COLS]
    return pl.pallas_call(
        body,
        out_shape=jax.ShapeDtypeStruct((m_per, n), jnp.float32),
        in_specs=[
            pl.BlockSpec(memory_space=pltpu.VMEM),
            pl.BlockSpec(memory_space=pltpu.VMEM),
            pl.BlockSpec(memory_space=pltpu.SMEM),
            pl.BlockSpec(memory_space=pltpu.SMEM),
        ],
        out_specs=pl.BlockSpec(memory_space=pltpu.VMEM),
        scratch_shapes=[
            *[pltpu.VMEM((N_DEV, m_per, wd), jnp.bfloat16) for wd in widths],
            *[pltpu.VMEM((4, m_per, wd), jnp.bfloat16) for wd in widths],
            *[pltpu.VMEM((2, m_per, wd), jnp.bfloat16) for wd in widths],
            *[pltpu.VMEM((m_per, wd), jnp.bfloat16) for wd in widths],
            pltpu.SemaphoreType.DMA((9,)),
            pltpu.SemaphoreType.DMA((9,)),
        ],
        compiler_params=pltpu.CompilerParams(
            collective_id=0, vmem_limit_bytes=100 * 1024 * 1024,
        ),
    )(x, w_mat, scale_x, scale_w)
